# baseline (speedup 1.0000x reference)
import functools

import jax
import jax.numpy as jnp
from jax import lax
from jax.experimental import pallas as pl
from jax.experimental.pallas import tpu as pltpu
from jax.experimental.pallas import tpu_sc as plsc

SEQ = 8192
DIM = 1024
NUM_CORES = 2
NUM_SUBCORES = 16
NW = NUM_CORES * NUM_SUBCORES
ROWS_PER_W = SEQ // NW
NB = 32
NBLK = ROWS_PER_W // NB


def _pos_embed_sc(batch, idx_hbm, table_hbm, out_hbm,
                  idx_v, buf0, buf1, gsem, wsem0, wsem1):
    c = lax.axis_index("c")
    s = lax.axis_index("s")
    wid = s * NUM_CORES + c
    base0 = wid * ROWS_PER_W

    bufs = (buf0, buf1)
    wsems = (wsem0, wsem1)

    pltpu.sync_copy(idx_hbm.at[wid], idx_v)

    pending = [None, None]
    for k in range(NBLK):
        p = k % 2
        if pending[p] is not None:
            for h in pending[p]:
                h.wait()
        pltpu.async_copy(table_hbm.at[idx_v.at[k]], bufs[p], gsem).wait()
        start = base0 + k * NB
        pending[p] = [
            pltpu.async_copy(bufs[p], out_hbm.at[pl.ds(b * SEQ + start, NB)],
                             wsems[p])
            for b in range(batch)
        ]
    for p in range(2):
        for h in pending[p]:
            h.wait()


def kernel(x, emb_weight, offset=0):
    seq = x.shape[-1]
    batch = 1
    for d in x.shape[:-1]:
        batch *= d
    off = jnp.asarray(offset, jnp.int32)
    positions = jnp.clip(jnp.arange(seq, dtype=jnp.int32) + off,
                         0, emb_weight.shape[0] - 1)
    positions = positions.reshape(NW, NBLK, NB)
    mesh = plsc.VectorSubcoreMesh(core_axis_name="c", subcore_axis_name="s")
    run = pl.kernel(
        functools.partial(_pos_embed_sc, batch),
        mesh=mesh,
        out_type=jax.ShapeDtypeStruct((batch * seq, DIM), jnp.float32),
        scratch_types=[
            pltpu.VMEM((NBLK, NB), jnp.int32),
            pltpu.VMEM((NB, DIM), jnp.float32),
            pltpu.VMEM((NB, DIM), jnp.float32),
            pltpu.SemaphoreType.DMA,
            pltpu.SemaphoreType.DMA,
            pltpu.SemaphoreType.DMA,
        ],
    )
    out = run(positions, emb_weight)
    return out.reshape(x.shape + (DIM,))


# speedup vs baseline: 1.0243x; 1.0243x over previous
import functools

import jax
import jax.numpy as jnp
from jax import lax
from jax.experimental import pallas as pl
from jax.experimental.pallas import tpu as pltpu
from jax.experimental.pallas import tpu_sc as plsc

SEQ = 8192
DIM = 1024
LANES = 16
NUM_CORES = 2
NUM_SUBCORES = 16
NW = NUM_CORES * NUM_SUBCORES
ROWS_PER_W = SEQ // NW
NB = 64
NBLK = ROWS_PER_W // NB


def _pos_embed_sc(batch, off_hbm, table_hbm, out_hbm,
                  off_v, idx_v, rows_v, sem, wsem):
    c = lax.axis_index("c")
    s = lax.axis_index("s")
    wid = s * NUM_CORES + c
    base0 = wid * ROWS_PER_W

    pltpu.sync_copy(off_hbm, off_v)
    off = off_v[...]
    lane = lax.iota(jnp.int32, LANES)
    for k in range(NBLK):
        for j in range(NB // LANES):
            base = base0 + k * NB + j * LANES
            pos = lane + off + base
            pos = lax.min(lax.max(pos, 0), SEQ - 1)
            idx_v[k, pl.ds(j * LANES, LANES)] = pos

    for k in range(NBLK):
        pltpu.async_copy(table_hbm.at[idx_v.at[k]], rows_v, sem).wait()
        start = base0 + k * NB
        writes = [
            pltpu.async_copy(rows_v, out_hbm.at[pl.ds(b * SEQ + start, NB)],
                             wsem)
            for b in range(batch)
        ]
        for h in writes:
            h.wait()


def kernel(x, emb_weight, offset=0):
    seq = x.shape[-1]
    batch = 1
    for d in x.shape[:-1]:
        batch *= d
    off16 = jnp.full((LANES,), jnp.asarray(offset, jnp.int32), jnp.int32)
    mesh = plsc.VectorSubcoreMesh(core_axis_name="c", subcore_axis_name="s")
    run = pl.kernel(
        functools.partial(_pos_embed_sc, batch),
        mesh=mesh,
        out_type=jax.ShapeDtypeStruct((batch * seq, DIM), jnp.float32),
        scratch_types=[
            pltpu.VMEM((LANES,), jnp.int32),
            pltpu.VMEM((NBLK, NB), jnp.int32),
            pltpu.VMEM((NB, DIM), jnp.float32),
            pltpu.SemaphoreType.DMA,
            pltpu.SemaphoreType.DMA,
        ],
    )
    out = run(off16, emb_weight)
    return out.reshape(x.shape + (DIM,))


# speedup vs baseline: 1.0291x; 1.0047x over previous
import functools

import jax
import jax.numpy as jnp
from jax import lax
from jax.experimental import pallas as pl
from jax.experimental.pallas import tpu as pltpu
from jax.experimental.pallas import tpu_sc as plsc

SEQ = 8192
DIM = 1024
LANES = 16
NUM_CORES = 2
NUM_SUBCORES = 16
NW = NUM_CORES * NUM_SUBCORES
ROWS_PER_W = SEQ // NW
NB = 64
NBLK = ROWS_PER_W // NB


def _pos_embed_sc(batch, off_hbm, table_hbm, out_hbm,
                  off_v, idx_v, rows_v, sem, wsem):
    c = lax.axis_index("c")
    s = lax.axis_index("s")
    wid = s * NUM_CORES + c
    base0 = wid * ROWS_PER_W

    pltpu.sync_copy(off_hbm, off_v)
    off = off_v[...]
    lane = lax.iota(jnp.int32, LANES)
    for k in range(NBLK):
        for j in range(NB // LANES):
            base = base0 + k * NB + j * LANES
            pos = lane + off + base
            pos = lax.min(lax.max(pos, 0), SEQ - 1)
            idx_v[k, pl.ds(j * LANES, LANES)] = pos

    for k in range(NBLK):
        pltpu.async_copy(table_hbm.at[idx_v.at[k]], rows_v, sem).wait()
        start = base0 + k * NB
        for b in range(batch):
            pltpu.sync_copy(rows_v, out_hbm.at[pl.ds(b * SEQ + start, NB)])


def kernel(x, emb_weight, offset=0):
    seq = x.shape[-1]
    batch = 1
    for d in x.shape[:-1]:
        batch *= d
    off16 = jnp.full((LANES,), jnp.asarray(offset, jnp.int32), jnp.int32)
    mesh = plsc.VectorSubcoreMesh(core_axis_name="c", subcore_axis_name="s")
    run = pl.kernel(
        functools.partial(_pos_embed_sc, batch),
        mesh=mesh,
        out_type=jax.ShapeDtypeStruct((batch * seq, DIM), jnp.float32),
        scratch_types=[
            pltpu.VMEM((LANES,), jnp.int32),
            pltpu.VMEM((NBLK, NB), jnp.int32),
            pltpu.VMEM((NB, DIM), jnp.float32),
            pltpu.SemaphoreType.DMA,
            pltpu.SemaphoreType.DMA,
        ],
    )
    out = run(off16, emb_weight)
    return out.reshape(x.shape + (DIM,))
